# baseline (speedup 1.0000x reference)
import functools

import jax
import jax.numpy as jnp
from jax.experimental import pallas as pl
from jax.experimental.pallas import tpu as pltpu

_NBUF = 16
_NOBUF = 4


def _prep_weights(w1t, w2r, b1r, b2_scalar, H):
    f32 = jnp.float32
    w_eff_row = jnp.dot(w2r.astype(f32), w1t.astype(f32),
                        preferred_element_type=f32)
    w_eff = jax.lax.transpose(w_eff_row, (1, 0))
    wa = w_eff[:H].reshape(1, H, 1)
    wb = w_eff[H:].reshape(1, H, 1)
    b_eff = jnp.sum(b1r.astype(f32) * w2r.astype(f32), axis=1,
                    keepdims=True) + b2_scalar
    return wa, wb, b_eff


def _chunk_compute(x1, x2, wa, wb, b_eff, out_dtype):
    s = x1 * wa + x2 * wb
    return (jnp.sum(s, axis=1) + b_eff).astype(out_dtype)


def _manual_kernel(f1_hbm, f2_hbm, w1t_ref, w2r_ref, b1_ref, b2_ref, o_hbm,
                   x1buf, x2buf, obuf, s1, s2, so, *, H, bc, nblk):
    core = pl.program_id(0)
    base = core * nblk

    def in_copies(k):
        slot = k % _NBUF
        g = base + k
        c1 = pltpu.make_async_copy(f1_hbm.at[pl.ds(g * bc, bc)],
                                   x1buf.at[slot], s1.at[slot])
        c2 = pltpu.make_async_copy(f2_hbm.at[pl.ds(g * bc, bc)],
                                   x2buf.at[slot], s2.at[slot])
        return c1, c2

    def out_copy(k):
        oslot = k % _NOBUF
        g = base + k
        return pltpu.make_async_copy(obuf.at[oslot],
                                     o_hbm.at[pl.ds(g * bc, bc)],
                                     so.at[oslot])

    for k in range(min(_NBUF - 1, nblk)):
        for c in in_copies(k):
            c.start()

    wa, wb, b_eff = _prep_weights(w1t_ref[...], w2r_ref[...], b1_ref[...],
                                  b2_ref[0, 0], H)

    for k in range(nblk):
        if k + _NBUF - 1 < nblk:
            for c in in_copies(k + _NBUF - 1):
                c.start()
        for c in in_copies(k):
            c.wait()
        if k >= _NOBUF:
            out_copy(k - _NOBUF).wait()
        slot = k % _NBUF
        obuf[k % _NOBUF] = _chunk_compute(x1buf[slot], x2buf[slot],
                                          wa, wb, b_eff, obuf.dtype)
        out_copy(k).start()

    for k in range(max(nblk - _NOBUF, 0), nblk):
        out_copy(k).wait()


def _emitter_kernel(f1_ref, f2_ref, w1t_ref, w2r_ref, b1_ref, b2_ref, o_ref,
                    *, H):
    wa, wb, b_eff = _prep_weights(w1t_ref[...], w2r_ref[...], b1_ref[...],
                                  b2_ref[0, 0], H)
    o_ref[...] = _chunk_compute(f1_ref[...], f2_ref[...], wa, wb, b_eff,
                                o_ref.dtype)


@functools.partial(jax.jit, static_argnames=("bc",))
def _mix_fusion(feat1, feat2, w1, b1, w2, b2, bc=16):
    B, T, H = feat1.shape
    out_dtype = feat1.dtype

    f1t = jnp.transpose(feat1, (0, 2, 1))
    f2t = jnp.transpose(feat2, (0, 2, 1))
    w1t = jnp.transpose(w1, (1, 0))
    w2r = w2.reshape(1, H)
    b1r = b1.reshape(1, H)
    b2r = b2.reshape(1, 1)

    weight_specs = [
        pl.BlockSpec((H, 2 * H), lambda i: (0, 0)),
        pl.BlockSpec((1, H), lambda i: (0, 0)),
        pl.BlockSpec((1, H), lambda i: (0, 0)),
        pl.BlockSpec(memory_space=pltpu.MemorySpace.SMEM),
    ]

    nblk_total = B // bc
    if B % bc == 0 and nblk_total % 2 == 0 and nblk_total >= 2 * _NBUF:
        nblk = nblk_total // 1
        out = pl.pallas_call(
            functools.partial(_manual_kernel, H=H, bc=bc, nblk=nblk),
            out_shape=jax.ShapeDtypeStruct((B, T), out_dtype),
            grid=(1,),
            in_specs=[
                pl.BlockSpec(memory_space=pltpu.MemorySpace.HBM),
                pl.BlockSpec(memory_space=pltpu.MemorySpace.HBM),
            ] + weight_specs,
            out_specs=pl.BlockSpec(memory_space=pltpu.MemorySpace.HBM),
            scratch_shapes=[
                pltpu.VMEM((_NBUF, bc, H, T), feat1.dtype),
                pltpu.VMEM((_NBUF, bc, H, T), feat2.dtype),
                pltpu.VMEM((_NOBUF, bc, T), out_dtype),
                pltpu.SemaphoreType.DMA((_NBUF,)),
                pltpu.SemaphoreType.DMA((_NBUF,)),
                pltpu.SemaphoreType.DMA((_NOBUF,)),
            ],
            compiler_params=pltpu.CompilerParams(
                dimension_semantics=("parallel",),
            ),
        )(f1t, f2t, w1t, w2r, b1r, b2r)
        return out

    bm = min(64, B)
    out = pl.pallas_call(
        functools.partial(_emitter_kernel, H=H),
        out_shape=jax.ShapeDtypeStruct((B, T), out_dtype),
        grid=(pl.cdiv(B, bm),),
        in_specs=[
            pl.BlockSpec((bm, H, T), lambda i: (i, 0, 0)),
            pl.BlockSpec((bm, H, T), lambda i: (i, 0, 0)),
        ] + weight_specs,
        out_specs=pl.BlockSpec((bm, T), lambda i: (i, 0)),
        compiler_params=pltpu.CompilerParams(
            dimension_semantics=("parallel",),
        ),
    )(f1t, f2t, w1t, w2r, b1r, b2r)
    return out


def kernel(feat1, feat2, score1, score2, w1, b1, w2, b2):
    del score1, score2
    return _mix_fusion(feat1, feat2, w1, b1, w2, b2)


# speedup vs baseline: 1.0550x; 1.0550x over previous
import functools

import jax
import jax.numpy as jnp
from jax.experimental import pallas as pl
from jax.experimental.pallas import tpu as pltpu

_NBUF = 8
_NOBUF = 4


def _prep_weights(w1t, w2r, b1r, b2_scalar, H):
    f32 = jnp.float32
    w_eff_row = jnp.dot(w2r.astype(f32), w1t.astype(f32),
                        preferred_element_type=f32)
    w_eff = jax.lax.transpose(w_eff_row, (1, 0))
    wa = w_eff[:H].reshape(1, H, 1)
    wb = w_eff[H:].reshape(1, H, 1)
    b_eff = jnp.sum(b1r.astype(f32) * w2r.astype(f32), axis=1,
                    keepdims=True) + b2_scalar
    return wa, wb, b_eff


def _chunk_compute(x1, x2, wa, wb, b_eff, out_dtype):
    s = x1 * wa + x2 * wb
    return (jnp.sum(s, axis=1) + b_eff).astype(out_dtype)


def _manual_kernel(f1_hbm, f2_hbm, w1t_ref, w2r_ref, b1_ref, b2_ref, o_hbm,
                   x1buf, x2buf, obuf, s1, s2, so, *, H, bc, nblk):
    core = pl.program_id(0)
    base = core * nblk

    def in_copies(k):
        slot = k % _NBUF
        g = base + k
        c1 = pltpu.make_async_copy(f1_hbm.at[pl.ds(g * bc, bc)],
                                   x1buf.at[slot], s1.at[slot])
        c2 = pltpu.make_async_copy(f2_hbm.at[pl.ds(g * bc, bc)],
                                   x2buf.at[slot], s2.at[slot])
        return c1, c2

    def out_copy(k):
        oslot = k % _NOBUF
        g = base + k
        return pltpu.make_async_copy(obuf.at[oslot],
                                     o_hbm.at[pl.ds(g * bc, bc)],
                                     so.at[oslot])

    for k in range(min(_NBUF - 1, nblk)):
        for c in in_copies(k):
            c.start()

    wa, wb, b_eff = _prep_weights(w1t_ref[...], w2r_ref[...], b1_ref[...],
                                  b2_ref[0, 0], H)

    for k in range(nblk):
        if k + _NBUF - 1 < nblk:
            for c in in_copies(k + _NBUF - 1):
                c.start()
        for c in in_copies(k):
            c.wait()
        if k >= _NOBUF:
            out_copy(k - _NOBUF).wait()
        slot = k % _NBUF
        obuf[k % _NOBUF] = _chunk_compute(x1buf[slot], x2buf[slot],
                                          wa, wb, b_eff, obuf.dtype)
        out_copy(k).start()

    for k in range(max(nblk - _NOBUF, 0), nblk):
        out_copy(k).wait()


def _emitter_kernel(f1_ref, f2_ref, w1t_ref, w2r_ref, b1_ref, b2_ref, o_ref,
                    *, H):
    wa, wb, b_eff = _prep_weights(w1t_ref[...], w2r_ref[...], b1_ref[...],
                                  b2_ref[0, 0], H)
    o_ref[...] = _chunk_compute(f1_ref[...], f2_ref[...], wa, wb, b_eff,
                                o_ref.dtype)


@functools.partial(jax.jit, static_argnames=("bc",))
def _mix_fusion(feat1, feat2, w1, b1, w2, b2, bc=32):
    B, T, H = feat1.shape
    out_dtype = feat1.dtype

    f1t = jnp.transpose(feat1, (0, 2, 1))
    f2t = jnp.transpose(feat2, (0, 2, 1))
    w1t = jnp.transpose(w1, (1, 0))
    w2r = w2.reshape(1, H)
    b1r = b1.reshape(1, H)
    b2r = b2.reshape(1, 1)

    weight_specs = [
        pl.BlockSpec((H, 2 * H), lambda i: (0, 0)),
        pl.BlockSpec((1, H), lambda i: (0, 0)),
        pl.BlockSpec((1, H), lambda i: (0, 0)),
        pl.BlockSpec(memory_space=pltpu.MemorySpace.SMEM),
    ]

    nblk_total = B // bc
    if B % bc == 0 and nblk_total % 2 == 0 and nblk_total >= 2 * _NBUF:
        nblk = nblk_total // 1
        out = pl.pallas_call(
            functools.partial(_manual_kernel, H=H, bc=bc, nblk=nblk),
            out_shape=jax.ShapeDtypeStruct((B, T), out_dtype),
            grid=(1,),
            in_specs=[
                pl.BlockSpec(memory_space=pltpu.MemorySpace.HBM),
                pl.BlockSpec(memory_space=pltpu.MemorySpace.HBM),
            ] + weight_specs,
            out_specs=pl.BlockSpec(memory_space=pltpu.MemorySpace.HBM),
            scratch_shapes=[
                pltpu.VMEM((_NBUF, bc, H, T), feat1.dtype),
                pltpu.VMEM((_NBUF, bc, H, T), feat2.dtype),
                pltpu.VMEM((_NOBUF, bc, T), out_dtype),
                pltpu.SemaphoreType.DMA((_NBUF,)),
                pltpu.SemaphoreType.DMA((_NBUF,)),
                pltpu.SemaphoreType.DMA((_NOBUF,)),
            ],
            compiler_params=pltpu.CompilerParams(
                dimension_semantics=("parallel",),
            ),
        )(f1t, f2t, w1t, w2r, b1r, b2r)
        return out

    bm = min(64, B)
    out = pl.pallas_call(
        functools.partial(_emitter_kernel, H=H),
        out_shape=jax.ShapeDtypeStruct((B, T), out_dtype),
        grid=(pl.cdiv(B, bm),),
        in_specs=[
            pl.BlockSpec((bm, H, T), lambda i: (i, 0, 0)),
            pl.BlockSpec((bm, H, T), lambda i: (i, 0, 0)),
        ] + weight_specs,
        out_specs=pl.BlockSpec((bm, T), lambda i: (i, 0)),
        compiler_params=pltpu.CompilerParams(
            dimension_semantics=("parallel",),
        ),
    )(f1t, f2t, w1t, w2r, b1r, b2r)
    return out


def kernel(feat1, feat2, score1, score2, w1, b1, w2, b2):
    del score1, score2
    return _mix_fusion(feat1, feat2, w1, b1, w2, b2)
